# baseline (speedup 1.0000x reference)
import functools

import jax
import jax.numpy as jnp
from jax import lax
from jax.experimental import pallas as pl
from jax.experimental.pallas import tpu as pltpu
from jax.experimental.pallas import tpu_sc as plsc

M, D = 200000, 512
B, R = 16384, 4096
NC, NS, L = 2, 16, 16
NW = NC * NS
CHUNK_M = 6256
CK = 128
OWN_MAX = R + CK


def _body(mem_hbm, memw_hbm, val_hbm, w_hbm, idx_hbm, ridx_hbm, out_hbm,
          pos_v, idx_v, ridx_v, riown_v, jown_v, rown_v,
          ric_v, jraw_v, jc_v, rowd_v, memwg_v, wupd_v, wt_v,
          rowc_v, jcc_v, mem_rows, sem_rows, sem_small, sem_idx, sem_fix):
    wid = lax.axis_index("s") * NC + lax.axis_index("c")
    base_m = wid * CHUNK_M
    iota = lax.broadcasted_iota(jnp.int32, (L,), 0)
    neg1 = jnp.full((L,), -1, jnp.int32)

    cp_idx = pltpu.async_copy(idx_hbm, idx_v, sem_idx)
    cp_ridx = pltpu.async_copy(ridx_hbm, ridx_v, sem_small)

    def fill(t, _):
        pos_v[pl.ds(t * L, L)] = neg1
        return 0

    lax.fori_loop(0, CHUNK_M // L, fill, 0)
    cp_idx.wait()

    def scan(t, _):
        iv = idx_v[pl.ds(t * L, L)]
        rel = iv - base_m
        mask = (rel >= 0) & (rel < CHUNK_M)
        relc = jnp.where(mask, rel, 0)
        plsc.store_scatter(pos_v, [relc], iota + t * L, mask=mask)
        return 0

    lax.fori_loop(0, B // L, scan, 0)
    cp_ridx.wait()

    def own(g, k):
        rv16 = ridx_v[pl.ds(g * L, L)]
        rel = rv16 - base_m
        mine = (rel >= 0) & (rel < CHUNK_M)
        relc = jnp.where(mine, rel, 0)
        j16 = plsc.load_gather(pos_v, [relc])
        dst = jnp.cumsum(jnp.where(mine, 1, 0)) - 1 + k
        dstc = jnp.where(mine, dst, 0)
        plsc.store_scatter(rown_v, [dstc], iota + g * L, mask=mine)
        plsc.store_scatter(riown_v, [dstc], rv16, mask=mine)
        plsc.store_scatter(jown_v, [dstc], j16, mask=mine)
        return k + jnp.sum(jnp.where(mine, 1, 0))

    k_own = lax.fori_loop(0, R // L, own, jnp.int32(0))

    neg_inf = jnp.int32(-2147483647)
    r0 = jnp.max(jnp.where(iota == 0, rown_v[pl.ds(0, L)], neg_inf))
    ri0 = jnp.max(jnp.where(iota == 0, riown_v[pl.ds(0, L)], neg_inf))
    j0 = jnp.max(jnp.where(iota == 0, jown_v[pl.ds(0, L)], neg_inf))
    for t in range(CK // L):
        rown_v[pl.ds(k_own + t * L, L)] = jnp.full((L,), 0, jnp.int32) + r0
        riown_v[pl.ds(k_own + t * L, L)] = jnp.full((L,), 0, jnp.int32) + ri0
        jown_v[pl.ds(k_own + t * L, L)] = jnp.full((L,), 0, jnp.int32) + j0

    n_chunks = (k_own + CK - 1) // CK

    def chunk(c, _):
        e0 = c * CK
        k_fix = jnp.int32(0)
        for g in range(CK // L):
            off = e0 + g * L
            ric_v[pl.ds(g * L, L)] = riown_v[pl.ds(off, L)]
            rowd_v[pl.ds(g * L, L)] = rown_v[pl.ds(off, L)]
            j16 = jown_v[pl.ds(off, L)]
            jraw_v[pl.ds(g * L, L)] = j16
            jc_v[pl.ds(g * L, L)] = jnp.where(j16 >= 0, j16, 0)

        cp_mem = pltpu.async_copy(mem_hbm.at[ric_v], mem_rows, sem_rows)
        cp_mw = pltpu.async_copy(memw_hbm.at[ric_v], memwg_v, sem_small)
        cp_w = pltpu.async_copy(w_hbm.at[jc_v], wupd_v, sem_small)

        for g in range(CK // L):
            j16 = jraw_v[pl.ds(g * L, L)]
            mask = j16 >= 0
            dst = jnp.cumsum(jnp.where(mask, 1, 0)) - 1 + k_fix
            dstc = jnp.where(mask, dst, 0)
            plsc.store_scatter(rowc_v, [dstc], iota + g * L, mask=mask)
            plsc.store_scatter(jcc_v, [dstc], j16, mask=mask)
            k_fix = k_fix + jnp.sum(jnp.where(mask, 1, 0))

        cp_mw.wait()
        cp_w.wait()
        for g in range(CK // L):
            mask = jraw_v[pl.ds(g * L, L)] >= 0
            wt_v[pl.ds(g * L, L)] = jnp.where(
                mask, wupd_v[pl.ds(g * L, L)], memwg_v[pl.ds(g * L, L)])
        cp_mem.wait()

        def fix(s, _):
            g0 = (s // L) * L
            lane = s - g0
            m = iota == lane
            jsc = jnp.max(jnp.where(m, jcc_v[pl.ds(g0, L)], neg_inf))
            rsc = jnp.max(jnp.where(m, rowc_v[pl.ds(g0, L)], neg_inf))
            pltpu.async_copy(val_hbm.at[jsc], mem_rows.at[rsc], sem_fix)
            return 0

        lax.fori_loop(0, k_fix, fix, 0)
        drain = pltpu.make_async_copy(val_hbm.at[0], mem_rows.at[0], sem_fix)

        def drain_one(s, _):
            drain.wait()
            return 0

        lax.fori_loop(0, k_fix, drain_one, 0)

        def rowgrp(g, _):
            wt16 = wt_v[pl.ds(g * L, L)]
            for s in range(L):
                i = g * L + s
                wt_s = wt16[s]
                for col in range(D // L):
                    mem_rows[i, pl.ds(col * L, L)] = (
                        mem_rows[i, pl.ds(col * L, L)] * wt_s)
            return 0

        lax.fori_loop(0, CK // L, rowgrp, 0)
        pltpu.sync_copy(mem_rows, out_hbm.at[rowd_v])
        return 0

    lax.fori_loop(0, n_chunks, chunk, 0)


@jax.jit
def _impl(mem, mem_w, val, w, idx, retrieve_idx):
    mesh = plsc.VectorSubcoreMesh(num_cores=NC, num_subcores=NS,
                                  core_axis_name="c", subcore_axis_name="s")
    params = pltpu.CompilerParams(needs_layout_passes=False)

    out = pl.kernel(
        _body,
        out_type=jax.ShapeDtypeStruct((R, D), jnp.float32),
        mesh=mesh,
        compiler_params=params,
        scratch_types=[
            pltpu.VMEM((CHUNK_M,), jnp.int32),
            pltpu.VMEM((B,), jnp.int32),
            pltpu.VMEM((R,), jnp.int32),
            pltpu.VMEM((OWN_MAX,), jnp.int32),
            pltpu.VMEM((OWN_MAX,), jnp.int32),
            pltpu.VMEM((OWN_MAX,), jnp.int32),
            pltpu.VMEM((CK,), jnp.int32),
            pltpu.VMEM((CK,), jnp.int32),
            pltpu.VMEM((CK,), jnp.int32),
            pltpu.VMEM((CK,), jnp.int32),
            pltpu.VMEM((CK,), jnp.float32),
            pltpu.VMEM((CK,), jnp.float32),
            pltpu.VMEM((CK,), jnp.float32),
            pltpu.VMEM((CK,), jnp.int32),
            pltpu.VMEM((CK,), jnp.int32),
            pltpu.VMEM((CK, D), jnp.float32),
            pltpu.SemaphoreType.DMA,
            pltpu.SemaphoreType.DMA,
            pltpu.SemaphoreType.DMA,
            pltpu.SemaphoreType.DMA,
        ],
    )(mem, mem_w, val, w, idx, retrieve_idx)
    return out


def kernel(mem, mem_w, val, w, idx, retrieve_idx):
    return _impl(mem, mem_w, val, w, idx, retrieve_idx)


# speedup vs baseline: 1.7073x; 1.7073x over previous
import functools

import jax
import jax.numpy as jnp
from jax import lax
from jax.experimental import pallas as pl
from jax.experimental.pallas import tpu as pltpu
from jax.experimental.pallas import tpu_sc as plsc

M, D = 200000, 512
B, R = 16384, 4096
NC, NS, L = 2, 16, 16
NW = NC * NS
CHUNK_M = 6256
M_PAD = NW * CHUNK_M
R_PER_W = R // NW

_MESH = dict(core_axis_name="c", subcore_axis_name="s")


def _wid():
    return lax.axis_index("s") * NC + lax.axis_index("c")


def _pos_body(idx_hbm, pos_hbm, pos_v, idx_v, sem_idx):
    wid = _wid()
    base_m = wid * CHUNK_M
    iota = lax.broadcasted_iota(jnp.int32, (L,), 0)
    neg1 = jnp.full((L,), -1, jnp.int32)
    cp_idx = pltpu.async_copy(idx_hbm, idx_v, sem_idx)

    def fill(t, _):
        pos_v[pl.ds(t * L, L)] = neg1
        return 0

    lax.fori_loop(0, CHUNK_M // L, fill, 0)
    cp_idx.wait()

    def scan(t, _):
        for u in range(2):
            t0 = t * 2 + u
            iv = idx_v[pl.ds(t0 * L, L)]
            rel = iv - base_m
            mask = (rel >= 0) & (rel < CHUNK_M)
            relc = jnp.where(mask, rel, 0)
            plsc.store_scatter(pos_v, [relc], iota + t0 * L, mask=mask)
        return 0

    lax.fori_loop(0, B // (2 * L), scan, 0)
    pltpu.sync_copy(pos_v, pos_hbm.at[pl.ds(base_m, CHUNK_M)])


def _retrieve_body(pos_hbm, mem_hbm, memw_hbm, val_hbm, w_hbm, ridx_hbm,
                   out_hbm, riv, posg_v, memwg_v, jc_v, wupd_v, wt_v,
                   rowc_v, jcc_v, mem_rows,
                   sem_rows, sem_small, sem_fix):
    wid = _wid()
    base_r = wid * R_PER_W
    iota = lax.broadcasted_iota(jnp.int32, (L,), 0)

    pltpu.sync_copy(ridx_hbm.at[pl.ds(base_r, R_PER_W)], riv)
    cp_mem = pltpu.async_copy(mem_hbm.at[riv], mem_rows, sem_rows)
    cp_pos = pltpu.async_copy(pos_hbm.at[riv], posg_v, sem_small)
    cp_mw = pltpu.async_copy(memw_hbm.at[riv], memwg_v, sem_small)
    cp_pos.wait()

    def clampj(g, _):
        j16 = posg_v[pl.ds(g * L, L)]
        jc_v[pl.ds(g * L, L)] = jnp.where(j16 >= 0, j16, 0)
        return 0

    lax.fori_loop(0, R_PER_W // L, clampj, 0)
    cp_w = pltpu.async_copy(w_hbm.at[jc_v], wupd_v, sem_small)
    cp_mw.wait()
    cp_w.wait()

    def compact(g, k):
        j16 = posg_v[pl.ds(g * L, L)]
        mask = j16 >= 0
        wt_v[pl.ds(g * L, L)] = jnp.where(
            mask, wupd_v[pl.ds(g * L, L)], memwg_v[pl.ds(g * L, L)])
        dst = jnp.cumsum(jnp.where(mask, 1, 0)) - 1 + k
        dstc = jnp.where(mask, dst, 0)
        plsc.store_scatter(rowc_v, [dstc], iota + g * L, mask=mask)
        plsc.store_scatter(jcc_v, [dstc], j16, mask=mask)
        return k + jnp.sum(jnp.where(mask, 1, 0))

    k_upd = lax.fori_loop(0, R_PER_W // L, compact, jnp.int32(0))
    cp_mem.wait()

    neg_inf = jnp.int32(-2147483647)

    def fix(s, _):
        g0 = (s // L) * L
        lane = s - g0
        m = iota == lane
        j16 = jcc_v[pl.ds(g0, L)]
        r16 = rowc_v[pl.ds(g0, L)]
        jsc = jnp.max(jnp.where(m, j16, neg_inf))
        rsc = jnp.max(jnp.where(m, r16, neg_inf))
        pltpu.async_copy(val_hbm.at[jsc], mem_rows.at[rsc], sem_fix)
        return 0

    lax.fori_loop(0, k_upd, fix, 0)
    drain = pltpu.make_async_copy(val_hbm.at[0], mem_rows.at[0], sem_fix)

    def drain_one(s, _):
        drain.wait()
        return 0

    lax.fori_loop(0, k_upd, drain_one, 0)

    def rowgrp(g, _):
        wt16 = wt_v[pl.ds(g * L, L)]
        for s in range(L):
            i = g * L + s
            wt_s = wt16[s]
            for c in range(D // L):
                mem_rows[i, pl.ds(c * L, L)] = (
                    mem_rows[i, pl.ds(c * L, L)] * wt_s)
        pltpu.async_copy(mem_rows.at[pl.ds(g * L, L)],
                         out_hbm.at[pl.ds(base_r + g * L, L)], sem_rows)
        return 0

    lax.fori_loop(0, R_PER_W // L, rowgrp, 0)
    wb = pltpu.make_async_copy(mem_rows.at[pl.ds(0, L)],
                               out_hbm.at[pl.ds(base_r, L)], sem_rows)

    def wb_drain(g, _):
        wb.wait()
        return 0

    lax.fori_loop(0, R_PER_W // L, wb_drain, 0)


@jax.jit
def _impl(mem, mem_w, val, w, idx, retrieve_idx):
    mesh = plsc.VectorSubcoreMesh(num_cores=NC, num_subcores=NS, **_MESH)
    params = pltpu.CompilerParams(needs_layout_passes=False)

    pos = pl.kernel(
        _pos_body,
        out_type=jax.ShapeDtypeStruct((M_PAD,), jnp.int32),
        mesh=mesh,
        compiler_params=params,
        scratch_types=[
            pltpu.VMEM((CHUNK_M,), jnp.int32),
            pltpu.VMEM((B,), jnp.int32),
            pltpu.SemaphoreType.DMA,
        ],
    )(idx)

    out = pl.kernel(
        _retrieve_body,
        out_type=jax.ShapeDtypeStruct((R, D), jnp.float32),
        mesh=mesh,
        compiler_params=params,
        scratch_types=[
            pltpu.VMEM((R_PER_W,), jnp.int32),
            pltpu.VMEM((R_PER_W,), jnp.int32),
            pltpu.VMEM((R_PER_W,), jnp.float32),
            pltpu.VMEM((R_PER_W,), jnp.int32),
            pltpu.VMEM((R_PER_W,), jnp.float32),
            pltpu.VMEM((R_PER_W,), jnp.float32),
            pltpu.VMEM((R_PER_W,), jnp.int32),
            pltpu.VMEM((R_PER_W,), jnp.int32),
            pltpu.VMEM((R_PER_W, D), jnp.float32),
            pltpu.SemaphoreType.DMA,
            pltpu.SemaphoreType.DMA,
            pltpu.SemaphoreType.DMA,
        ],
    )(pos, mem, mem_w, val, w, retrieve_idx)
    return out


def kernel(mem, mem_w, val, w, idx, retrieve_idx):
    return _impl(mem, mem_w, val, w, idx, retrieve_idx)


# speedup vs baseline: 1.7076x; 1.0002x over previous
import functools

import jax
import jax.numpy as jnp
from jax import lax
from jax.experimental import pallas as pl
from jax.experimental.pallas import tpu as pltpu
from jax.experimental.pallas import tpu_sc as plsc

M, D = 200000, 512
B, R = 16384, 4096
NC, NS, L = 2, 16, 16
NW = NC * NS
CHUNK_M = 6256
M_PAD = NW * CHUNK_M
R_PER_W = R // NW

_MESH = dict(core_axis_name="c", subcore_axis_name="s")


def _wid():
    return lax.axis_index("s") * NC + lax.axis_index("c")


def _pos_body(idx_hbm, pos_hbm, pos_v, idx_v, sem_idx):
    wid = _wid()
    base_m = wid * CHUNK_M
    iota = lax.broadcasted_iota(jnp.int32, (L,), 0)
    neg1 = jnp.full((L,), -1, jnp.int32)
    cp_idx = pltpu.async_copy(idx_hbm, idx_v, sem_idx)

    def fill(t, _):
        pos_v[pl.ds(t * L, L)] = neg1
        return 0

    lax.fori_loop(0, CHUNK_M // L, fill, 0)
    cp_idx.wait()

    def scan(t, _):
        for u in range(2):
            t0 = t * 2 + u
            iv = idx_v[pl.ds(t0 * L, L)]
            rel = iv - base_m
            mask = (rel >= 0) & (rel < CHUNK_M)
            relc = jnp.where(mask, rel, 0)
            plsc.store_scatter(pos_v, [relc], iota + t0 * L, mask=mask)
        return 0

    lax.fori_loop(0, B // (2 * L), scan, 0)
    pltpu.sync_copy(pos_v, pos_hbm.at[pl.ds(base_m, CHUNK_M)])


def _retrieve_body(pos_hbm, mem_hbm, memw_hbm, val_hbm, w_hbm, ridx_hbm,
                   out_hbm, riv, posg_v, memwg_v, jc_v, wupd_v, wt_v,
                   rowc_v, jcc_v, mem_rows,
                   sem_rows, sem_small, sem_fix):
    wid = _wid()
    base_r = wid * R_PER_W
    iota = lax.broadcasted_iota(jnp.int32, (L,), 0)

    pltpu.sync_copy(ridx_hbm.at[pl.ds(base_r, R_PER_W)], riv)
    cp_mem = pltpu.async_copy(mem_hbm.at[riv], mem_rows, sem_rows)
    cp_pos = pltpu.async_copy(pos_hbm.at[riv], posg_v, sem_small)
    cp_mw = pltpu.async_copy(memw_hbm.at[riv], memwg_v, sem_small)
    cp_pos.wait()

    def clampj(g, _):
        j16 = posg_v[pl.ds(g * L, L)]
        jc_v[pl.ds(g * L, L)] = jnp.where(j16 >= 0, j16, 0)
        return 0

    lax.fori_loop(0, R_PER_W // L, clampj, 0)
    cp_w = pltpu.async_copy(w_hbm.at[jc_v], wupd_v, sem_small)
    cp_mw.wait()
    cp_w.wait()

    def compact(g, k):
        j16 = posg_v[pl.ds(g * L, L)]
        mask = j16 >= 0
        wt_v[pl.ds(g * L, L)] = jnp.where(
            mask, wupd_v[pl.ds(g * L, L)], memwg_v[pl.ds(g * L, L)])
        dst = jnp.cumsum(jnp.where(mask, 1, 0)) - 1 + k
        dstc = jnp.where(mask, dst, 0)
        plsc.store_scatter(rowc_v, [dstc], iota + g * L, mask=mask)
        plsc.store_scatter(jcc_v, [dstc], j16, mask=mask)
        return k + jnp.sum(jnp.where(mask, 1, 0))

    k_upd = lax.fori_loop(0, R_PER_W // L, compact, jnp.int32(0))
    cp_mem.wait()

    neg_inf = jnp.int32(-2147483647)

    def fix(s, _):
        g0 = (s // L) * L
        lane = s - g0
        m = iota == lane
        j16 = jcc_v[pl.ds(g0, L)]
        r16 = rowc_v[pl.ds(g0, L)]
        jsc = jnp.max(jnp.where(m, j16, neg_inf))
        rsc = jnp.max(jnp.where(m, r16, neg_inf))
        pltpu.async_copy(val_hbm.at[jsc], mem_rows.at[rsc], sem_fix)
        return 0

    lax.fori_loop(0, k_upd, fix, 0)
    drain = pltpu.make_async_copy(val_hbm.at[0], mem_rows.at[0], sem_fix)

    def drain_one(s, _):
        drain.wait()
        return 0

    lax.fori_loop(0, k_upd, drain_one, 0)

    def rowgrp(g, _):
        wt16 = wt_v[pl.ds(g * L, L)]
        for s in range(L):
            i = g * L + s
            wt_s = wt16[s]
            for c in range(D // L):
                mem_rows[i, pl.ds(c * L, L)] = (
                    mem_rows[i, pl.ds(c * L, L)] * wt_s)
        pltpu.async_copy(mem_rows.at[pl.ds(g * L, L)],
                         out_hbm.at[pl.ds(base_r + g * L, L)], sem_rows)
        return 0

    lax.fori_loop(0, R_PER_W // L, rowgrp, 0)
    wb = pltpu.make_async_copy(mem_rows.at[pl.ds(0, L)],
                               out_hbm.at[pl.ds(base_r, L)], sem_rows)

    def wb_drain(g, _):
        wb.wait()
        return 0

    lax.fori_loop(0, R_PER_W // L, wb_drain, 0)


@jax.jit
def _impl(mem, mem_w, val, w, idx, retrieve_idx):
    mesh = plsc.VectorSubcoreMesh(num_cores=NC, num_subcores=NS, **_MESH)
    params = pltpu.CompilerParams(needs_layout_passes=False,
                                  skip_device_barrier=True)

    pos = pl.kernel(
        _pos_body,
        out_type=jax.ShapeDtypeStruct((M_PAD,), jnp.int32),
        mesh=mesh,
        compiler_params=params,
        scratch_types=[
            pltpu.VMEM((CHUNK_M,), jnp.int32),
            pltpu.VMEM((B,), jnp.int32),
            pltpu.SemaphoreType.DMA,
        ],
    )(idx)

    out = pl.kernel(
        _retrieve_body,
        out_type=jax.ShapeDtypeStruct((R, D), jnp.float32),
        mesh=mesh,
        compiler_params=params,
        scratch_types=[
            pltpu.VMEM((R_PER_W,), jnp.int32),
            pltpu.VMEM((R_PER_W,), jnp.int32),
            pltpu.VMEM((R_PER_W,), jnp.float32),
            pltpu.VMEM((R_PER_W,), jnp.int32),
            pltpu.VMEM((R_PER_W,), jnp.float32),
            pltpu.VMEM((R_PER_W,), jnp.float32),
            pltpu.VMEM((R_PER_W,), jnp.int32),
            pltpu.VMEM((R_PER_W,), jnp.int32),
            pltpu.VMEM((R_PER_W, D), jnp.float32),
            pltpu.SemaphoreType.DMA,
            pltpu.SemaphoreType.DMA,
            pltpu.SemaphoreType.DMA,
        ],
    )(pos, mem, mem_w, val, w, retrieve_idx)
    return out


def kernel(mem, mem_w, val, w, idx, retrieve_idx):
    return _impl(mem, mem_w, val, w, idx, retrieve_idx)


# speedup vs baseline: 2.2915x; 1.3420x over previous
import functools

import jax
import jax.numpy as jnp
from jax import lax
from jax.experimental import pallas as pl
from jax.experimental.pallas import tpu as pltpu
from jax.experimental.pallas import tpu_sc as plsc

M, D = 200000, 512
B, R = 16384, 4096
NC, NS, L = 2, 16, 16
NW = NC * NS
CHUNK_M = 6256
M_PAD = NW * CHUNK_M
R_PER_W = R // NW

_MESH = dict(core_axis_name="c", subcore_axis_name="s")


def _wid():
    return lax.axis_index("s") * NC + lax.axis_index("c")


def _pos_body(idx_hbm, pos_hbm, pos_v, idx_v, sem_idx):
    wid = _wid()
    base_m = wid * CHUNK_M
    iota = lax.broadcasted_iota(jnp.int32, (L,), 0)
    neg1 = jnp.full((L,), -1, jnp.int32)
    cp_idx = pltpu.async_copy(idx_hbm, idx_v, sem_idx)

    def fill(t, _):
        pos_v[pl.ds(t * L, L)] = neg1
        return 0

    lax.fori_loop(0, CHUNK_M // L, fill, 0)
    cp_idx.wait()

    def scan(t, _):
        for u in range(2):
            t0 = t * 2 + u
            iv = idx_v[pl.ds(t0 * L, L)]
            rel = iv - base_m
            mask = (rel >= 0) & (rel < CHUNK_M)
            relc = jnp.where(mask, rel, 0)
            plsc.store_scatter(pos_v, [relc], iota + t0 * L, mask=mask)
        return 0

    lax.fori_loop(0, B // (2 * L), scan, 0)
    pltpu.sync_copy(pos_v, pos_hbm.at[pl.ds(base_m, CHUNK_M)])


def _retrieve_body(pos_hbm, mem_hbm, memw_hbm, val_hbm, w_hbm, ridx_hbm,
                   out_hbm, riv, posg_v, memwg_v, w_v, wt_v,
                   rowc_v, jcc_v, mem_rows,
                   sem_rows, sem_small, sem_w, sem_fix):
    wid = _wid()
    base_r = wid * R_PER_W
    iota = lax.broadcasted_iota(jnp.int32, (L,), 0)

    cp_wall = pltpu.async_copy(w_hbm, w_v, sem_w)
    pltpu.sync_copy(ridx_hbm.at[pl.ds(base_r, R_PER_W)], riv)
    cp_mem = pltpu.async_copy(mem_hbm.at[riv], mem_rows, sem_rows)
    cp_pos = pltpu.async_copy(pos_hbm.at[riv], posg_v, sem_small)
    cp_mw = pltpu.async_copy(memw_hbm.at[riv], memwg_v, sem_small)
    cp_pos.wait()
    cp_mw.wait()
    cp_wall.wait()

    def compact(g, k):
        j16 = posg_v[pl.ds(g * L, L)]
        mask = j16 >= 0
        jc16 = jnp.where(mask, j16, 0)
        wupd = plsc.load_gather(w_v, [jc16])
        wt_v[pl.ds(g * L, L)] = jnp.where(
            mask, wupd, memwg_v[pl.ds(g * L, L)])
        dst = jnp.cumsum(jnp.where(mask, 1, 0)) - 1 + k
        dstc = jnp.where(mask, dst, 0)
        plsc.store_scatter(rowc_v, [dstc], iota + g * L, mask=mask)
        plsc.store_scatter(jcc_v, [dstc], j16, mask=mask)
        return k + jnp.sum(jnp.where(mask, 1, 0))

    k_upd = lax.fori_loop(0, R_PER_W // L, compact, jnp.int32(0))
    cp_mem.wait()

    neg_inf = jnp.int32(-2147483647)

    def fix(s, _):
        g0 = (s // L) * L
        lane = s - g0
        m = iota == lane
        j16 = jcc_v[pl.ds(g0, L)]
        r16 = rowc_v[pl.ds(g0, L)]
        jsc = jnp.max(jnp.where(m, j16, neg_inf))
        rsc = jnp.max(jnp.where(m, r16, neg_inf))
        pltpu.async_copy(val_hbm.at[jsc], mem_rows.at[rsc], sem_fix)
        return 0

    lax.fori_loop(0, k_upd, fix, 0)
    drain = pltpu.make_async_copy(val_hbm.at[0], mem_rows.at[0], sem_fix)

    def drain_one(s, _):
        drain.wait()
        return 0

    lax.fori_loop(0, k_upd, drain_one, 0)

    def rowgrp(g, _):
        wt16 = wt_v[pl.ds(g * L, L)]
        for s in range(L):
            i = g * L + s
            wt_s = wt16[s]
            for c in range(D // L):
                mem_rows[i, pl.ds(c * L, L)] = (
                    mem_rows[i, pl.ds(c * L, L)] * wt_s)
        pltpu.async_copy(mem_rows.at[pl.ds(g * L, L)],
                         out_hbm.at[pl.ds(base_r + g * L, L)], sem_rows)
        return 0

    lax.fori_loop(0, R_PER_W // L, rowgrp, 0)
    wb = pltpu.make_async_copy(mem_rows.at[pl.ds(0, L)],
                               out_hbm.at[pl.ds(base_r, L)], sem_rows)

    def wb_drain(g, _):
        wb.wait()
        return 0

    lax.fori_loop(0, R_PER_W // L, wb_drain, 0)


@jax.jit
def _impl(mem, mem_w, val, w, idx, retrieve_idx):
    mesh = plsc.VectorSubcoreMesh(num_cores=NC, num_subcores=NS, **_MESH)
    params = pltpu.CompilerParams(needs_layout_passes=False)

    pos = pl.kernel(
        _pos_body,
        out_type=jax.ShapeDtypeStruct((M_PAD,), jnp.int32),
        mesh=mesh,
        compiler_params=params,
        scratch_types=[
            pltpu.VMEM((CHUNK_M,), jnp.int32),
            pltpu.VMEM((B,), jnp.int32),
            pltpu.SemaphoreType.DMA,
        ],
    )(idx)

    out = pl.kernel(
        _retrieve_body,
        out_type=jax.ShapeDtypeStruct((R, D), jnp.float32),
        mesh=mesh,
        compiler_params=params,
        scratch_types=[
            pltpu.VMEM((R_PER_W,), jnp.int32),
            pltpu.VMEM((R_PER_W,), jnp.int32),
            pltpu.VMEM((R_PER_W,), jnp.float32),
            pltpu.VMEM((B,), jnp.float32),
            pltpu.VMEM((R_PER_W,), jnp.float32),
            pltpu.VMEM((R_PER_W,), jnp.int32),
            pltpu.VMEM((R_PER_W,), jnp.int32),
            pltpu.VMEM((R_PER_W, D), jnp.float32),
            pltpu.SemaphoreType.DMA,
            pltpu.SemaphoreType.DMA,
            pltpu.SemaphoreType.DMA,
            pltpu.SemaphoreType.DMA,
        ],
    )(pos, mem, mem_w, val, w, retrieve_idx)
    return out


def kernel(mem, mem_w, val, w, idx, retrieve_idx):
    return _impl(mem, mem_w, val, w, idx, retrieve_idx)
